# Initial kernel scaffold; baseline (speedup 1.0000x reference)
import jax
import jax.numpy as jnp
from jax.experimental import pallas as pl


def kernel(pcl_noisy, pcl_clean, params, pnt_idx):
    raise NotImplementedError("write your pallas kernel here")


# speedup vs baseline: 8.8112x; 8.8112x over previous
import functools

import jax
import jax.numpy as jnp
from jax import lax
from jax.experimental import pallas as pl
from jax.experimental.pallas import tpu as pltpu
from jax.experimental.pallas import tpu_sc as plsc

_NC, _NS, _L = 2, 16, 16
_NW = _NC * _NS

_B, _N, _Q = 4, 10000, 512
_QPW = (_B * _Q) // _NW
_SLOTS = _Q // _QPW
_NCH = _N // _L
_GROUP = 5
_NGRP = _NCH // _GROUP
_CAP = 4096
_CAP2 = 1024
_OUTW = 16

_BIG = 3e38


def _knn_body(noisy_hbm, clean_hbm, idx_hbm, out_hbm,
              px, py, pz, cx, cy, cz, idx_v, cand_d, cand_i, b2d, b2i, out_v):
    wid = lax.axis_index("s") * _NC + lax.axis_index("c")
    b = wid // _SLOTS
    slot = wid % _SLOTS
    base = b * 3 * _N
    pltpu.sync_copy(noisy_hbm.at[pl.ds(base, _N)], px)
    pltpu.sync_copy(noisy_hbm.at[pl.ds(base + _N, _N)], py)
    pltpu.sync_copy(noisy_hbm.at[pl.ds(base + 2 * _N, _N)], pz)
    pltpu.sync_copy(clean_hbm.at[pl.ds(base, _N)], cx)
    pltpu.sync_copy(clean_hbm.at[pl.ds(base + _N, _N)], cy)
    pltpu.sync_copy(clean_hbm.at[pl.ds(base + 2 * _N, _N)], cz)
    pltpu.sync_copy(idx_hbm.at[pl.ds(slot * _QPW, _QPW)],
                    idx_v.at[pl.ds(0, _QPW)])

    iota = lax.iota(jnp.int32, _L)
    zf = jnp.zeros((_L,), jnp.float32)
    zi = jnp.zeros((_L,), jnp.int32)

    def select_sum(xr, yr, zr, qx, qy, qz, K, depth):
        def grp(g, carry):
            m1, m2, off, tb = carry
            for k in range(_GROUP):
                c = g * _GROUP + k
                sl = pl.ds(c * _L, _L)
                dx = xr[sl] - qx
                d = dx * dx
                dy = yr[sl] - qy
                d = d + dy * dy
                dz = zr[sl] - qz
                d = d + dz * dz
                if depth == 2:
                    m2 = jnp.minimum(m2, jnp.maximum(m1, d))
                m1 = jnp.minimum(m1, d)
                msk = d <= tb
                cnt = plsc.cumsum(msk.astype(jnp.int32))
                pos = jnp.minimum(off + cnt - 1, _CAP - 1)
                plsc.store_scatter(cand_d, [pos], d, mask=msk)
                plsc.store_scatter(cand_i, [pos], c * _L + iota, mask=msk)
                off = off + plsc.all_reduce_population_count(msk)
            tb = jnp.full((_L,), jnp.max(m2 if depth == 2 else m1))
            return m1, m2, off, tb
        init = (jnp.full((_L,), _BIG, jnp.float32), jnp.full((_L,), _BIG, jnp.float32), zi,
                jnp.full((_L,), _BIG, jnp.float32))
        m1, m2, off, _ = lax.fori_loop(0, _NGRP, grp, init)

        tex = jnp.full((_L,), jnp.max(m2 if depth == 2 else m1))
        c_tot = jnp.minimum(jnp.max(off), _CAP)
        c_tot_s = jnp.full((_L,), c_tot)
        nch = (c_tot + _L - 1) // _L

        def shrink(cc, off2):
            sl = pl.ds(cc * _L, _L)
            d = cand_d[sl]
            vi = cand_i[sl]
            msk = (d <= tex) & ((cc * _L + iota) < c_tot_s)
            cnt = plsc.cumsum(msk.astype(jnp.int32))
            pos = jnp.minimum(off2 + cnt - 1, _CAP2 - 1)
            plsc.store_scatter(b2d, [pos], d, mask=msk)
            plsc.store_scatter(b2i, [pos], vi, mask=msk)
            return off2 + plsc.all_reduce_population_count(msk)
        off2 = lax.fori_loop(0, nch, shrink, zi)
        c2 = jnp.minimum(jnp.max(off2), _CAP2)
        c2s = jnp.full((_L,), c2)
        nch2 = (c2 + _L - 1) // _L

        def bit_step(i, carry):
            prefix, remaining = carry
            bit = jnp.left_shift(jnp.int32(1), 30 - i)
            keep = jnp.full((_L,), -bit)
            bits = jnp.full((_L,), bit)

            def cntb(cc, acc):
                vb = plsc.bitcast(b2d[pl.ds(cc * _L, _L)], jnp.int32)
                hit = ((vb & keep) == prefix) & ((cc * _L + iota) < c2s)
                return acc + plsc.all_reduce_population_count(hit)
            cnt = lax.fori_loop(0, nch2, cntb, zi)
            take1 = cnt < remaining
            prefix = jnp.where(take1, prefix | bits, prefix)
            remaining = jnp.where(take1, remaining - cnt, remaining)
            return prefix, remaining
        prefix, remaining = lax.fori_loop(
            0, 31, bit_step, (zi, jnp.full((_L,), K, jnp.int32)))

        def sum_step(cc, carry):
            sx, sy, sz, got = carry
            sl = pl.ds(cc * _L, _L)
            vb = plsc.bitcast(b2d[sl], jnp.int32)
            vi = b2i[sl]
            valid = (cc * _L + iota) < c2s
            lt = (vb < prefix) & valid
            eq = (vb == prefix) & valid
            cinc = plsc.cumsum(eq.astype(jnp.int32))
            sel = eq & ((got + cinc) <= remaining)
            m = lt | sel
            gx = plsc.load_gather(xr, [vi], mask=m)
            gy = plsc.load_gather(yr, [vi], mask=m)
            gz = plsc.load_gather(zr, [vi], mask=m)
            sx = sx + jnp.where(m, gx, zf)
            sy = sy + jnp.where(m, gy, zf)
            sz = sz + jnp.where(m, gz, zf)
            got = got + plsc.all_reduce_population_count(sel)
            return sx, sy, sz, got
        sx, sy, sz, _ = lax.fori_loop(0, nch2, sum_step, (zf, zf, zf, zi))
        return jnp.sum(sx), jnp.sum(sy), jnp.sum(sz)

    def qbody(j, carry):
        qidx = plsc.load_gather(idx_v, [jnp.full((_L,), j, jnp.int32)])
        qx = plsc.load_gather(px, [qidx])
        qy = plsc.load_gather(py, [qidx])
        qz = plsc.load_gather(pz, [qidx])
        nx, ny, nz = select_sum(px, py, pz, qx, qy, qz, 32, 2)
        ex, ey, ez = select_sum(cx, cy, cz, qx, qy, qz, 4, 1)
        out = jnp.where(iota == 0, qx, zf)
        out = jnp.where(iota == 1, qy, out)
        out = jnp.where(iota == 2, qz, out)
        out = jnp.where(iota == 3, nx, out)
        out = jnp.where(iota == 4, ny, out)
        out = jnp.where(iota == 5, nz, out)
        out = jnp.where(iota == 6, ex, out)
        out = jnp.where(iota == 7, ey, out)
        out = jnp.where(iota == 8, ez, out)
        out_v[pl.ds(j * _OUTW, _OUTW)] = out
        return carry
    lax.fori_loop(0, _QPW, qbody, 0)
    pltpu.sync_copy(out_v, out_hbm.at[pl.ds(wid * _QPW * _OUTW, _QPW * _OUTW)])


_knn_call = functools.partial(
    pl.kernel,
    out_type=jax.ShapeDtypeStruct((_NW * _QPW * _OUTW,), jnp.float32),
    mesh=plsc.VectorSubcoreMesh(core_axis_name="c", subcore_axis_name="s",
                                num_cores=_NC, num_subcores=_NS),
    compiler_params=pltpu.CompilerParams(needs_layout_passes=False),
    scratch_types=[
        pltpu.VMEM((_N,), jnp.float32),
        pltpu.VMEM((_N,), jnp.float32),
        pltpu.VMEM((_N,), jnp.float32),
        pltpu.VMEM((_N,), jnp.float32),
        pltpu.VMEM((_N,), jnp.float32),
        pltpu.VMEM((_N,), jnp.float32),
        pltpu.VMEM((max(_QPW, 128),), jnp.int32),
        pltpu.VMEM((_CAP,), jnp.float32),
        pltpu.VMEM((_CAP,), jnp.int32),
        pltpu.VMEM((_CAP2,), jnp.float32),
        pltpu.VMEM((_CAP2,), jnp.int32),
        pltpu.VMEM((_QPW * _OUTW,), jnp.float32),
    ],
)(_knn_body)


def _mlp_body(sc_ref, w1_ref, b1_ref, w2_ref, b2_ref, afc_ref, anv_ref,
              wfc_ref, wft_ref, bin_ref, wb_ref, bb_ref, wo_ref, bo_ref,
              out_ref):
    f32 = jnp.float32

    def dot(a, bm):
        return lax.dot_general(a, bm, (((1,), (0,)), ((), ())),
                               preferred_element_type=f32)
    sc = sc_ref[...]
    h = jnp.maximum(dot(sc, w1_ref[...]) + b1_ref[...], 0.0)
    feat = dot(h, w2_ref[...]) + b2_ref[...]
    fc = dot(sc, afc_ref[...])
    nv = dot(sc, anv_ref[...])
    s = jnp.maximum(dot(fc, wfc_ref[...]) + dot(feat, wft_ref[...])
                    + bin_ref[...], 0.0)
    for i in range(4):
        s = jnp.maximum(dot(s, wb_ref[i]) + bb_ref[i][None, :], 0.0) + s
    g = dot(s, wo_ref[...]) + bo_ref[...]
    diff = nv + g
    out_ref[...] = (jnp.sum(diff * diff) * f32(0.5 * 100.0 / (_B * _Q))
                    ).reshape(1, 1)


def kernel(pcl_noisy, pcl_clean, params, pnt_idx):
    f32 = jnp.float32
    noisy_flat = jnp.transpose(pcl_noisy, (0, 2, 1)).reshape(-1)
    clean_flat = jnp.transpose(pcl_clean, (0, 2, 1)).reshape(-1)
    sc = _knn_call(noisy_flat, clean_flat,
                   pnt_idx.astype(jnp.int32)).reshape(_B * _Q, _OUTW)

    p = params
    w1 = jnp.zeros((_OUTW, 64), f32).at[0:3].set(p['fW1'])
    b1 = p['fb1'].reshape(1, 64)
    w2 = p['fW2']
    b2 = p['fb2'].reshape(1, 128)
    afc = (jnp.zeros((_OUTW, _OUTW), f32)
           .at[0, 0].set(-1.0).at[1, 1].set(-1.0).at[2, 2].set(-1.0)
           .at[3, 0].set(1.0 / 32).at[4, 1].set(1.0 / 32)
           .at[5, 2].set(1.0 / 32))
    anv = (jnp.zeros((_OUTW, _OUTW), f32)
           .at[0, 0].set(1.0).at[1, 1].set(1.0).at[2, 2].set(1.0)
           .at[6, 0].set(-0.25).at[7, 1].set(-0.25).at[8, 2].set(-0.25))
    wfc = jnp.zeros((_OUTW, 128), f32).at[0:3].set(p['sWin'][0:3])
    wft = p['sWin'][3:]
    bin_ = p['sbin'].reshape(1, 128)
    wb = jnp.stack(p['sWb'])
    bb = jnp.stack(p['sbb'])
    wo = jnp.zeros((128, _OUTW), f32).at[:, 0:3].set(p['sWout'])
    bo = jnp.zeros((1, _OUTW), f32).at[0, 0:3].set(p['sbout'])

    loss = pl.pallas_call(
        _mlp_body,
        out_shape=jax.ShapeDtypeStruct((1, 1), f32),
    )(sc, w1, b1, w2, b2, afc, anv, wfc, wft, bin_, wb, bb, wo, bo)
    return loss[0, 0]


# speedup vs baseline: 11.7602x; 1.3347x over previous
import functools

import jax
import jax.numpy as jnp
from jax import lax
from jax.experimental import pallas as pl
from jax.experimental.pallas import tpu as pltpu
from jax.experimental.pallas import tpu_sc as plsc

_NC, _NS, _L = 2, 16, 16
_NW = _NC * _NS

_B, _N, _Q = 4, 10000, 512
_QPW = (_B * _Q) // _NW
_SLOTS = _Q // _QPW
_NCH = _N // _L
_GROUP = 5
_NGRP = _NCH // _GROUP
_S = 128
_S2 = 32
_OUTW = 16

_BIG = 3e38


def _knn_body(noisy_hbm, clean_hbm, idx_hbm, out_hbm,
              px, py, pz, cx, cy, cz, idx_v, cand_d, cand_i, b2d, b2i, out_v):
    wid = lax.axis_index("s") * _NC + lax.axis_index("c")
    b = wid // _SLOTS
    slot = wid % _SLOTS
    base = b * 3 * _N
    pltpu.sync_copy(noisy_hbm.at[pl.ds(base, _N)], px)
    pltpu.sync_copy(noisy_hbm.at[pl.ds(base + _N, _N)], py)
    pltpu.sync_copy(noisy_hbm.at[pl.ds(base + 2 * _N, _N)], pz)
    pltpu.sync_copy(clean_hbm.at[pl.ds(base, _N)], cx)
    pltpu.sync_copy(clean_hbm.at[pl.ds(base + _N, _N)], cy)
    pltpu.sync_copy(clean_hbm.at[pl.ds(base + 2 * _N, _N)], cz)
    pltpu.sync_copy(idx_hbm.at[pl.ds(slot * _QPW, _QPW)],
                    idx_v.at[pl.ds(0, _QPW)])

    iota = lax.iota(jnp.int32, _L)
    zf = jnp.zeros((_L,), jnp.float32)
    zi = jnp.zeros((_L,), jnp.int32)

    def select_sum(xr, yr, zr, qx, qy, qz, K, depth):
        def grp(g, carry):
            m1, m2, offl, tb = carry
            for k in range(_GROUP):
                c = g * _GROUP + k
                sl = pl.ds(c * _L, _L)
                dx = xr[sl] - qx
                d = dx * dx
                dy = yr[sl] - qy
                d = d + dy * dy
                dz = zr[sl] - qz
                d = d + dz * dz
                if depth == 2:
                    m2 = jnp.minimum(m2, jnp.maximum(m1, d))
                m1 = jnp.minimum(m1, d)
                msk = d <= tb
                pos = jnp.minimum(offl, _S - 1) * _L + iota
                plsc.store_scatter(cand_d, [pos], d, mask=msk)
                plsc.store_scatter(cand_i, [pos], c * _L + iota, mask=msk)
                offl = offl + msk.astype(jnp.int32)
            tb = jnp.full((_L,), jnp.max(m2 if depth == 2 else m1))
            return m1, m2, offl, tb
        init = (jnp.full((_L,), _BIG, jnp.float32),
                jnp.full((_L,), _BIG, jnp.float32), zi,
                jnp.full((_L,), _BIG, jnp.float32))
        m1, m2, offl, _ = lax.fori_loop(0, _NGRP, grp, init)

        tex = jnp.full((_L,), jnp.max(m2 if depth == 2 else m1))
        rows = jnp.minimum(jnp.max(offl), _S)

        def shrink(r, off2):
            sl = pl.ds(r * _L, _L)
            d = cand_d[sl]
            vi = cand_i[sl]
            msk = (d <= tex) & (offl > r)
            pos = jnp.minimum(off2, _S2 - 1) * _L + iota
            plsc.store_scatter(b2d, [pos], d, mask=msk)
            plsc.store_scatter(b2i, [pos], vi, mask=msk)
            return off2 + msk.astype(jnp.int32)
        off2 = lax.fori_loop(0, rows, shrink, zi)
        rows2 = jnp.minimum(jnp.max(off2), _S2)

        def bit_step(i, carry):
            prefix, remaining = carry
            bit = jnp.left_shift(jnp.int32(1), 30 - i)
            keep = jnp.full((_L,), -bit)
            bits = jnp.full((_L,), bit)

            def cntb(r, acc):
                vb = plsc.bitcast(b2d[pl.ds(r * _L, _L)], jnp.int32)
                hit = ((vb & keep) == prefix) & (off2 > r)
                return acc + plsc.all_reduce_population_count(hit)
            cnt = lax.fori_loop(0, rows2, cntb, zi)
            take1 = cnt < remaining
            prefix = jnp.where(take1, prefix | bits, prefix)
            remaining = jnp.where(take1, remaining - cnt, remaining)
            return prefix, remaining
        prefix, remaining = lax.fori_loop(
            0, 31, bit_step, (zi, jnp.full((_L,), K, jnp.int32)))

        def sum_step(r, carry):
            sx, sy, sz, got = carry
            sl = pl.ds(r * _L, _L)
            vb = plsc.bitcast(b2d[sl], jnp.int32)
            vi = b2i[sl]
            valid = off2 > r
            lt = (vb < prefix) & valid
            eq = (vb == prefix) & valid
            cinc = plsc.cumsum(eq.astype(jnp.int32))
            sel = eq & ((got + cinc) <= remaining)
            m = lt | sel
            gx = plsc.load_gather(xr, [vi], mask=m)
            gy = plsc.load_gather(yr, [vi], mask=m)
            gz = plsc.load_gather(zr, [vi], mask=m)
            sx = sx + jnp.where(m, gx, zf)
            sy = sy + jnp.where(m, gy, zf)
            sz = sz + jnp.where(m, gz, zf)
            got = got + plsc.all_reduce_population_count(sel)
            return sx, sy, sz, got
        sx, sy, sz, _ = lax.fori_loop(0, rows2, sum_step, (zf, zf, zf, zi))
        return jnp.sum(sx), jnp.sum(sy), jnp.sum(sz)

    def qbody(j, carry):
        qidx = plsc.load_gather(idx_v, [jnp.full((_L,), j, jnp.int32)])
        qx = plsc.load_gather(px, [qidx])
        qy = plsc.load_gather(py, [qidx])
        qz = plsc.load_gather(pz, [qidx])
        nx, ny, nz = select_sum(px, py, pz, qx, qy, qz, 32, 2)
        ex, ey, ez = select_sum(cx, cy, cz, qx, qy, qz, 4, 1)
        out = jnp.where(iota == 0, qx, zf)
        out = jnp.where(iota == 1, qy, out)
        out = jnp.where(iota == 2, qz, out)
        out = jnp.where(iota == 3, nx, out)
        out = jnp.where(iota == 4, ny, out)
        out = jnp.where(iota == 5, nz, out)
        out = jnp.where(iota == 6, ex, out)
        out = jnp.where(iota == 7, ey, out)
        out = jnp.where(iota == 8, ez, out)
        out_v[pl.ds(j * _OUTW, _OUTW)] = out
        return carry
    lax.fori_loop(0, _QPW, qbody, 0)
    pltpu.sync_copy(out_v, out_hbm.at[pl.ds(wid * _QPW * _OUTW, _QPW * _OUTW)])


_knn_call = functools.partial(
    pl.kernel,
    out_type=jax.ShapeDtypeStruct((_NW * _QPW * _OUTW,), jnp.float32),
    mesh=plsc.VectorSubcoreMesh(core_axis_name="c", subcore_axis_name="s",
                                num_cores=_NC, num_subcores=_NS),
    compiler_params=pltpu.CompilerParams(needs_layout_passes=False),
    scratch_types=[
        pltpu.VMEM((_N,), jnp.float32),
        pltpu.VMEM((_N,), jnp.float32),
        pltpu.VMEM((_N,), jnp.float32),
        pltpu.VMEM((_N,), jnp.float32),
        pltpu.VMEM((_N,), jnp.float32),
        pltpu.VMEM((_N,), jnp.float32),
        pltpu.VMEM((max(_QPW, 128),), jnp.int32),
        pltpu.VMEM((_S * _L,), jnp.float32),
        pltpu.VMEM((_S * _L,), jnp.int32),
        pltpu.VMEM((_S2 * _L,), jnp.float32),
        pltpu.VMEM((_S2 * _L,), jnp.int32),
        pltpu.VMEM((_QPW * _OUTW,), jnp.float32),
    ],
)(_knn_body)


def _mlp_body(sc_ref, w1_ref, b1_ref, w2_ref, b2_ref, afc_ref, anv_ref,
              wfc_ref, wft_ref, bin_ref, wb_ref, bb_ref, wo_ref, bo_ref,
              out_ref):
    f32 = jnp.float32

    def dot(a, bm):
        return lax.dot_general(a, bm, (((1,), (0,)), ((), ())),
                               preferred_element_type=f32)
    sc = sc_ref[...]
    h = jnp.maximum(dot(sc, w1_ref[...]) + b1_ref[...], 0.0)
    feat = dot(h, w2_ref[...]) + b2_ref[...]
    fc = dot(sc, afc_ref[...])
    nv = dot(sc, anv_ref[...])
    s = jnp.maximum(dot(fc, wfc_ref[...]) + dot(feat, wft_ref[...])
                    + bin_ref[...], 0.0)
    for i in range(4):
        s = jnp.maximum(dot(s, wb_ref[i]) + bb_ref[i][None, :], 0.0) + s
    g = dot(s, wo_ref[...]) + bo_ref[...]
    diff = nv + g
    out_ref[...] = (jnp.sum(diff * diff) * f32(0.5 * 100.0 / (_B * _Q))
                    ).reshape(1, 1)


def kernel(pcl_noisy, pcl_clean, params, pnt_idx):
    f32 = jnp.float32
    noisy_flat = jnp.transpose(pcl_noisy, (0, 2, 1)).reshape(-1)
    clean_flat = jnp.transpose(pcl_clean, (0, 2, 1)).reshape(-1)
    sc = _knn_call(noisy_flat, clean_flat,
                   pnt_idx.astype(jnp.int32)).reshape(_B * _Q, _OUTW)

    p = params
    w1 = jnp.zeros((_OUTW, 64), f32).at[0:3].set(p['fW1'])
    b1 = p['fb1'].reshape(1, 64)
    w2 = p['fW2']
    b2 = p['fb2'].reshape(1, 128)
    afc = (jnp.zeros((_OUTW, _OUTW), f32)
           .at[0, 0].set(-1.0).at[1, 1].set(-1.0).at[2, 2].set(-1.0)
           .at[3, 0].set(1.0 / 32).at[4, 1].set(1.0 / 32)
           .at[5, 2].set(1.0 / 32))
    anv = (jnp.zeros((_OUTW, _OUTW), f32)
           .at[0, 0].set(1.0).at[1, 1].set(1.0).at[2, 2].set(1.0)
           .at[6, 0].set(-0.25).at[7, 1].set(-0.25).at[8, 2].set(-0.25))
    wfc = jnp.zeros((_OUTW, 128), f32).at[0:3].set(p['sWin'][0:3])
    wft = p['sWin'][3:]
    bin_ = p['sbin'].reshape(1, 128)
    wb = jnp.stack(p['sWb'])
    bb = jnp.stack(p['sbb'])
    wo = jnp.zeros((128, _OUTW), f32).at[:, 0:3].set(p['sWout'])
    bo = jnp.zeros((1, _OUTW), f32).at[0, 0:3].set(p['sbout'])

    loss = pl.pallas_call(
        _mlp_body,
        out_shape=jax.ShapeDtypeStruct((1, 1), f32),
    )(sc, w1, b1, w2, b2, afc, anv, wfc, wft, bin_, wb, bb, wo, bo)
    return loss[0, 0]


# speedup vs baseline: 26.0550x; 2.2155x over previous
import functools

import jax
import jax.numpy as jnp
from jax import lax
from jax.experimental import pallas as pl
from jax.experimental.pallas import tpu as pltpu
from jax.experimental.pallas import tpu_sc as plsc

_NC, _NS, _L = 2, 16, 16
_NW = _NC * _NS

_B, _N, _Q = 4, 10000, 512
_QPW = (_B * _Q) // _NW
_SLOTS = _Q // _QPW
_NCH = _N // _L
_GROUP = 5
_NGRP = _NCH // _GROUP
_NQB = 4
_S = 128
_S2 = 32
_S2PAD = _S2 + 4
_OUTW = 16

_BIG = 3e38


def _knn_body(noisy_hbm, clean_hbm, idx_hbm, out_hbm,
              px, py, pz, cx, cy, cz, idx_v, cand_d, cand_i, b2d, b2i, out_v):
    wid = lax.axis_index("s") * _NC + lax.axis_index("c")
    b = wid // _SLOTS
    slot = wid % _SLOTS
    base = b * 3 * _N
    pltpu.sync_copy(noisy_hbm.at[pl.ds(base, _N)], px)
    pltpu.sync_copy(noisy_hbm.at[pl.ds(base + _N, _N)], py)
    pltpu.sync_copy(noisy_hbm.at[pl.ds(base + 2 * _N, _N)], pz)
    pltpu.sync_copy(clean_hbm.at[pl.ds(base, _N)], cx)
    pltpu.sync_copy(clean_hbm.at[pl.ds(base + _N, _N)], cy)
    pltpu.sync_copy(clean_hbm.at[pl.ds(base + 2 * _N, _N)], cz)
    pltpu.sync_copy(idx_hbm.at[pl.ds(slot * _QPW, _QPW)],
                    idx_v.at[pl.ds(0, _QPW)])

    iota = lax.iota(jnp.int32, _L)
    zf = jnp.zeros((_L,), jnp.float32)
    zi = jnp.zeros((_L,), jnp.int32)
    climit = (_S - 1) * _L + iota

    def _shuf(v, s):
        return v.at[iota ^ s].get(mode="promise_in_bounds")

    def vmaxs(v):
        for s in (8, 4, 2, 1):
            v = jnp.maximum(v, _shuf(v, s))
        return v

    def vadds(v):
        for s in (8, 4, 2, 1):
            v = v + _shuf(v, s)
        return v

    def scan4(xr, yr, zr, qxs, qys, qzs, depth):
        def grp(g, carry):
            m1s, m2s, offp, tbs = carry
            m1s, m2s, offp, tbs = list(m1s), list(m2s), list(offp), list(tbs)
            for k in range(_GROUP):
                c = g * _GROUP + k
                sl = pl.ds(c * _L, _L)
                vx = xr[sl]
                vy = yr[sl]
                vz = zr[sl]
                ci = c * _L + iota
                for qq in range(_NQB):
                    dx = vx - qxs[qq]
                    d = dx * dx
                    dy = vy - qys[qq]
                    d = d + dy * dy
                    dz = vz - qzs[qq]
                    d = d + dz * dz
                    if depth == 2:
                        m2s[qq] = jnp.minimum(m2s[qq],
                                              jnp.maximum(m1s[qq], d))
                    m1s[qq] = jnp.minimum(m1s[qq], d)
                    msk = d <= tbs[qq]
                    pos = qq * (_S * _L) + jnp.minimum(offp[qq], climit)
                    plsc.store_scatter(cand_d, [pos], d, mask=msk)
                    plsc.store_scatter(cand_i, [pos], ci, mask=msk)
                    offp[qq] = offp[qq] + jnp.where(msk, 16, 0)
            for qq in range(_NQB):
                tbs[qq] = vmaxs(m2s[qq] if depth == 2 else m1s[qq])
            return tuple(m1s), tuple(m2s), tuple(offp), tuple(tbs)

        big = jnp.full((_L,), _BIG, jnp.float32)
        init = ((big,) * _NQB, (big,) * _NQB, (iota,) * _NQB, (big,) * _NQB)
        m1s, m2s, offp, _ = lax.fori_loop(0, _NGRP, grp, init)
        msrc = m2s if depth == 2 else m1s
        offls = tuple((offp[qq] - iota) >> 4 for qq in range(_NQB))
        return msrc, offls

    def finish(qq, msrc, offl, xr, yr, zr, K):
        tex = vmaxs(msrc)
        rows = jnp.minimum(jnp.max(offl), _S)
        base1 = qq * (_S * _L)

        def shrink(r, off2):
            sl = pl.ds(base1 + r * _L, _L)
            d = cand_d[sl]
            vi = cand_i[sl]
            msk = (d <= tex) & (offl > r)
            pos = jnp.minimum(off2, _S2 - 1) * _L + iota
            plsc.store_scatter(b2d, [pos], d, mask=msk)
            plsc.store_scatter(b2i, [pos], vi, mask=msk)
            return off2 + msk.astype(jnp.int32)
        off2 = lax.fori_loop(0, rows, shrink, zi)
        off2 = jnp.minimum(off2, _S2)
        rows2 = jnp.max(off2)
        nb4 = (rows2 + 3) >> 2

        def bit_step(i, carry):
            prefix, remaining = carry
            bit = jnp.left_shift(jnp.int32(1), 30 - i)
            keep = jnp.full((_L,), -bit)
            bits = jnp.full((_L,), bit)

            def cntb(r4, acc):
                for u in range(4):
                    r = r4 * 4 + u
                    vb = plsc.bitcast(b2d[pl.ds(r * _L, _L)], jnp.int32)
                    hit = ((vb & keep) == prefix) & (off2 > r)
                    acc = acc + plsc.all_reduce_population_count(hit)
                return acc
            cnt = lax.fori_loop(0, nb4, cntb, zi)
            take1 = cnt < remaining
            prefix = jnp.where(take1, prefix | bits, prefix)
            remaining = jnp.where(take1, remaining - cnt, remaining)
            return prefix, remaining
        prefix, remaining = lax.fori_loop(
            0, 31, bit_step, (zi, jnp.full((_L,), K, jnp.int32)))

        def sum_step(r, carry):
            sx, sy, sz, got = carry
            sl = pl.ds(r * _L, _L)
            vb = plsc.bitcast(b2d[sl], jnp.int32)
            vi = b2i[sl]
            valid = off2 > r
            lt = (vb < prefix) & valid
            eq = (vb == prefix) & valid
            cinc = plsc.cumsum(eq.astype(jnp.int32))
            sel = eq & ((got + cinc) <= remaining)
            m = lt | sel
            gx = plsc.load_gather(xr, [vi], mask=m)
            gy = plsc.load_gather(yr, [vi], mask=m)
            gz = plsc.load_gather(zr, [vi], mask=m)
            sx = sx + jnp.where(m, gx, zf)
            sy = sy + jnp.where(m, gy, zf)
            sz = sz + jnp.where(m, gz, zf)
            got = got + plsc.all_reduce_population_count(sel)
            return sx, sy, sz, got
        sx, sy, sz, _ = lax.fori_loop(0, rows2, sum_step, (zf, zf, zf, zi))
        return vadds(sx), vadds(sy), vadds(sz)

    def qgroup(jg, carry):
        j0 = jg * _NQB
        qxs, qys, qzs = [], [], []
        for qq in range(_NQB):
            qidx = plsc.load_gather(
                idx_v, [jnp.full((_L,), j0 + qq, jnp.int32)])
            qxs.append(plsc.load_gather(px, [qidx]))
            qys.append(plsc.load_gather(py, [qidx]))
            qzs.append(plsc.load_gather(pz, [qidx]))
        nsum = []
        msrc, offls = scan4(px, py, pz, qxs, qys, qzs, 2)
        for qq in range(_NQB):
            nsum.append(finish(qq, msrc[qq], offls[qq], px, py, pz, 32))
        esum = []
        msrc, offls = scan4(cx, cy, cz, qxs, qys, qzs, 1)
        for qq in range(_NQB):
            esum.append(finish(qq, msrc[qq], offls[qq], cx, cy, cz, 4))
        for qq in range(_NQB):
            out = jnp.where(iota == 0, qxs[qq], zf)
            out = jnp.where(iota == 1, qys[qq], out)
            out = jnp.where(iota == 2, qzs[qq], out)
            out = jnp.where(iota == 3, nsum[qq][0], out)
            out = jnp.where(iota == 4, nsum[qq][1], out)
            out = jnp.where(iota == 5, nsum[qq][2], out)
            out = jnp.where(iota == 6, esum[qq][0], out)
            out = jnp.where(iota == 7, esum[qq][1], out)
            out = jnp.where(iota == 8, esum[qq][2], out)
            out_v[pl.ds((j0 + qq) * _OUTW, _OUTW)] = out
        return carry
    lax.fori_loop(0, _QPW // _NQB, qgroup, 0)
    pltpu.sync_copy(out_v, out_hbm.at[pl.ds(wid * _QPW * _OUTW, _QPW * _OUTW)])


_knn_call = functools.partial(
    pl.kernel,
    out_type=jax.ShapeDtypeStruct((_NW * _QPW * _OUTW,), jnp.float32),
    mesh=plsc.VectorSubcoreMesh(core_axis_name="c", subcore_axis_name="s",
                                num_cores=_NC, num_subcores=_NS),
    compiler_params=pltpu.CompilerParams(needs_layout_passes=False),
    scratch_types=[
        pltpu.VMEM((_N,), jnp.float32),
        pltpu.VMEM((_N,), jnp.float32),
        pltpu.VMEM((_N,), jnp.float32),
        pltpu.VMEM((_N,), jnp.float32),
        pltpu.VMEM((_N,), jnp.float32),
        pltpu.VMEM((_N,), jnp.float32),
        pltpu.VMEM((max(_QPW, 128),), jnp.int32),
        pltpu.VMEM((_NQB * _S * _L,), jnp.float32),
        pltpu.VMEM((_NQB * _S * _L,), jnp.int32),
        pltpu.VMEM((_S2PAD * _L,), jnp.float32),
        pltpu.VMEM((_S2PAD * _L,), jnp.int32),
        pltpu.VMEM((_QPW * _OUTW,), jnp.float32),
    ],
)(_knn_body)


def _mlp_body(sc_ref, w1_ref, b1_ref, w2_ref, b2_ref, afc_ref, anv_ref,
              wfc_ref, wft_ref, bin_ref, wb_ref, bb_ref, wo_ref, bo_ref,
              out_ref):
    f32 = jnp.float32

    def dot(a, bm):
        return lax.dot_general(a, bm, (((1,), (0,)), ((), ())),
                               preferred_element_type=f32)
    sc = sc_ref[...]
    h = jnp.maximum(dot(sc, w1_ref[...]) + b1_ref[...], 0.0)
    feat = dot(h, w2_ref[...]) + b2_ref[...]
    fc = dot(sc, afc_ref[...])
    nv = dot(sc, anv_ref[...])
    s = jnp.maximum(dot(fc, wfc_ref[...]) + dot(feat, wft_ref[...])
                    + bin_ref[...], 0.0)
    for i in range(4):
        s = jnp.maximum(dot(s, wb_ref[i]) + bb_ref[i][None, :], 0.0) + s
    g = dot(s, wo_ref[...]) + bo_ref[...]
    diff = nv + g
    out_ref[...] = (jnp.sum(diff * diff) * f32(0.5 * 100.0 / (_B * _Q))
                    ).reshape(1, 1)


def kernel(pcl_noisy, pcl_clean, params, pnt_idx):
    f32 = jnp.float32
    noisy_flat = jnp.transpose(pcl_noisy, (0, 2, 1)).reshape(-1)
    clean_flat = jnp.transpose(pcl_clean, (0, 2, 1)).reshape(-1)
    sc = _knn_call(noisy_flat, clean_flat,
                   pnt_idx.astype(jnp.int32)).reshape(_B * _Q, _OUTW)

    p = params
    w1 = jnp.zeros((_OUTW, 64), f32).at[0:3].set(p['fW1'])
    b1 = p['fb1'].reshape(1, 64)
    w2 = p['fW2']
    b2 = p['fb2'].reshape(1, 128)
    afc = (jnp.zeros((_OUTW, _OUTW), f32)
           .at[0, 0].set(-1.0).at[1, 1].set(-1.0).at[2, 2].set(-1.0)
           .at[3, 0].set(1.0 / 32).at[4, 1].set(1.0 / 32)
           .at[5, 2].set(1.0 / 32))
    anv = (jnp.zeros((_OUTW, _OUTW), f32)
           .at[0, 0].set(1.0).at[1, 1].set(1.0).at[2, 2].set(1.0)
           .at[6, 0].set(-0.25).at[7, 1].set(-0.25).at[8, 2].set(-0.25))
    wfc = jnp.zeros((_OUTW, 128), f32).at[0:3].set(p['sWin'][0:3])
    wft = p['sWin'][3:]
    bin_ = p['sbin'].reshape(1, 128)
    wb = jnp.stack(p['sWb'])
    bb = jnp.stack(p['sbb'])
    wo = jnp.zeros((128, _OUTW), f32).at[:, 0:3].set(p['sWout'])
    bo = jnp.zeros((1, _OUTW), f32).at[0, 0:3].set(p['sbout'])

    loss = pl.pallas_call(
        _mlp_body,
        out_shape=jax.ShapeDtypeStruct((1, 1), f32),
    )(sc, w1, b1, w2, b2, afc, anv, wfc, wft, bin_, wb, bb, wo, bo)
    return loss[0, 0]


# speedup vs baseline: 32.8897x; 1.2623x over previous
import functools

import jax
import jax.numpy as jnp
from jax import lax
from jax.experimental import pallas as pl
from jax.experimental.pallas import tpu as pltpu
from jax.experimental.pallas import tpu_sc as plsc

_NC, _NS, _L = 2, 16, 16
_NW = _NC * _NS

_B, _N, _Q = 4, 10000, 512
_QPW = (_B * _Q) // _NW
_SLOTS = _Q // _QPW
_NCH = _N // _L
_GROUP = 5
_NGRP = _NCH // _GROUP
_NQB = 4
_S = 128
_S2 = 32
_S2PAD = _S2 + 4
_OUTW = 16

_BIG = 3e38


def _knn_body(noisy_hbm, clean_hbm, idx_hbm, out_hbm,
              px, py, pz, cx, cy, cz, idx_v, cand_d, cand_i, b2d, b2i, out_v):
    wid = lax.axis_index("s") * _NC + lax.axis_index("c")
    b = wid // _SLOTS
    slot = wid % _SLOTS
    base = b * 3 * _N
    pltpu.sync_copy(noisy_hbm.at[pl.ds(base, _N)], px)
    pltpu.sync_copy(noisy_hbm.at[pl.ds(base + _N, _N)], py)
    pltpu.sync_copy(noisy_hbm.at[pl.ds(base + 2 * _N, _N)], pz)
    pltpu.sync_copy(clean_hbm.at[pl.ds(base, _N)], cx)
    pltpu.sync_copy(clean_hbm.at[pl.ds(base + _N, _N)], cy)
    pltpu.sync_copy(clean_hbm.at[pl.ds(base + 2 * _N, _N)], cz)
    pltpu.sync_copy(idx_hbm.at[pl.ds(slot * _QPW, _QPW)],
                    idx_v.at[pl.ds(0, _QPW)])

    iota = lax.iota(jnp.int32, _L)
    zf = jnp.zeros((_L,), jnp.float32)
    zi = jnp.zeros((_L,), jnp.int32)
    climit = (_S - 1) * _L + iota

    def _shuf(v, s):
        return v.at[iota ^ s].get(mode="promise_in_bounds")

    def vmaxs(v):
        for s in (8, 4, 2, 1):
            v = jnp.maximum(v, _shuf(v, s))
        return v

    def vadds(v):
        for s in (8, 4, 2, 1):
            v = v + _shuf(v, s)
        return v

    def scan4(xr, yr, zr, qxs, qys, qzs, depth):
        def grp(g, carry):
            m1s, m2s, offp, tbs = carry
            m1s, m2s, offp, tbs = list(m1s), list(m2s), list(offp), list(tbs)
            for k in range(_GROUP):
                c = g * _GROUP + k
                sl = pl.ds(c * _L, _L)
                vx = xr[sl]
                vy = yr[sl]
                vz = zr[sl]
                ci = c * _L + iota
                for qq in range(_NQB):
                    dx = vx - qxs[qq]
                    d = dx * dx
                    dy = vy - qys[qq]
                    d = d + dy * dy
                    dz = vz - qzs[qq]
                    d = d + dz * dz
                    if depth == 2:
                        m2s[qq] = jnp.minimum(m2s[qq],
                                              jnp.maximum(m1s[qq], d))
                    m1s[qq] = jnp.minimum(m1s[qq], d)
                    msk = d <= tbs[qq]
                    pos = qq * (_S * _L) + jnp.minimum(offp[qq], climit)
                    plsc.store_scatter(cand_d, [pos], d, mask=msk)
                    plsc.store_scatter(cand_i, [pos], ci, mask=msk)
                    offp[qq] = offp[qq] + jnp.where(msk, 16, 0)
            for qq in range(_NQB):
                tbs[qq] = vmaxs(m2s[qq] if depth == 2 else m1s[qq])
            return tuple(m1s), tuple(m2s), tuple(offp), tuple(tbs)

        big = jnp.full((_L,), _BIG, jnp.float32)
        init = ((big,) * _NQB, (big,) * _NQB, (iota,) * _NQB, (big,) * _NQB)
        m1s, m2s, offp, _ = lax.fori_loop(0, _NGRP, grp, init)
        msrc = m2s if depth == 2 else m1s
        offls = tuple((offp[qq] - iota) >> 4 for qq in range(_NQB))
        return msrc, offls

    def finish4(msrcs, offls, xr, yr, zr, K):
        texs = [vmaxs(msrcs[qq]) for qq in range(_NQB)]
        rmax = jnp.maximum(jnp.maximum(offls[0], offls[1]),
                           jnp.maximum(offls[2], offls[3]))
        rows = jnp.minimum(jnp.max(rmax), _S)

        def shrink(r, off2s):
            off2s = list(off2s)
            for qq in range(_NQB):
                sl = pl.ds(qq * (_S * _L) + r * _L, _L)
                d = cand_d[sl]
                vi = cand_i[sl]
                msk = (d <= texs[qq]) & (offls[qq] > r)
                pos = (qq * (_S2PAD * _L)
                       + jnp.minimum(off2s[qq], _S2 - 1) * _L + iota)
                plsc.store_scatter(b2d, [pos], d, mask=msk)
                plsc.store_scatter(b2i, [pos], vi, mask=msk)
                off2s[qq] = off2s[qq] + msk.astype(jnp.int32)
            return tuple(off2s)
        off2s = lax.fori_loop(0, rows, shrink, (zi,) * _NQB)
        off2s = [jnp.minimum(o, _S2) for o in off2s]
        rows2 = jnp.max(jnp.maximum(jnp.maximum(off2s[0], off2s[1]),
                                    jnp.maximum(off2s[2], off2s[3])))

        big = jnp.full((_L,), _BIG, jnp.float32)
        if K == 32:
            def fold(r, carry):
                new = []
                for qq in range(_NQB):
                    lok, lov, hik, hiv = carry[qq]
                    sl = pl.ds(qq * (_S2PAD * _L) + r * _L, _L)
                    kd = jnp.where(off2s[qq] > r, b2d[sl], big)
                    sk, sv = plsc.sort_key_val(kd, b2i[sl])
                    rsk = _shuf(sk, 15)
                    rsv = _shuf(sv, 15)
                    m = hik <= rsk
                    wk = jnp.where(m, hik, rsk)
                    wv = jnp.where(m, hiv, rsv)
                    wk, wv = plsc.sort_key_val(wk, wv)
                    rwk = _shuf(wk, 15)
                    rwv = _shuf(wv, 15)
                    m2 = lok <= rwk
                    nlk = jnp.where(m2, lok, rwk)
                    nlv = jnp.where(m2, lov, rwv)
                    nhk = jnp.where(m2, rwk, lok)
                    nhv = jnp.where(m2, rwv, lov)
                    nlk, nlv = plsc.sort_key_val(nlk, nlv)
                    nhk, nhv = plsc.sort_key_val(nhk, nhv)
                    new.append((nlk, nlv, nhk, nhv))
                return tuple(new)
            st = lax.fori_loop(0, rows2, fold, ((big, zi, big, zi),) * _NQB)
            sums = []
            for qq in range(_NQB):
                _, lov, _, hiv = st[qq]
                gx = plsc.load_gather(xr, [lov]) + plsc.load_gather(xr, [hiv])
                gy = plsc.load_gather(yr, [lov]) + plsc.load_gather(yr, [hiv])
                gz = plsc.load_gather(zr, [lov]) + plsc.load_gather(zr, [hiv])
                sums.append((vadds(gx), vadds(gy), vadds(gz)))
            return sums
        else:
            def fold(r, carry):
                new = []
                for qq in range(_NQB):
                    lok, lov = carry[qq]
                    sl = pl.ds(qq * (_S2PAD * _L) + r * _L, _L)
                    kd = jnp.where(off2s[qq] > r, b2d[sl], big)
                    sk, sv = plsc.sort_key_val(kd, b2i[sl])
                    rsk = _shuf(sk, 15)
                    rsv = _shuf(sv, 15)
                    m = lok <= rsk
                    wk = jnp.where(m, lok, rsk)
                    wv = jnp.where(m, lov, rsv)
                    lok, lov = plsc.sort_key_val(wk, wv)
                    new.append((lok, lov))
                return tuple(new)
            st = lax.fori_loop(0, rows2, fold, ((big, zi),) * _NQB)
            mk = iota < K
            sums = []
            for qq in range(_NQB):
                _, lov = st[qq]
                gx = plsc.load_gather(xr, [lov], mask=mk)
                gy = plsc.load_gather(yr, [lov], mask=mk)
                gz = plsc.load_gather(zr, [lov], mask=mk)
                sums.append((vadds(jnp.where(mk, gx, zf)),
                             vadds(jnp.where(mk, gy, zf)),
                             vadds(jnp.where(mk, gz, zf))))
            return sums

    def qgroup(jg, carry):
        j0 = jg * _NQB
        qxs, qys, qzs = [], [], []
        for qq in range(_NQB):
            qidx = plsc.load_gather(
                idx_v, [jnp.full((_L,), j0 + qq, jnp.int32)])
            qxs.append(plsc.load_gather(px, [qidx]))
            qys.append(plsc.load_gather(py, [qidx]))
            qzs.append(plsc.load_gather(pz, [qidx]))
        msrc, offls = scan4(px, py, pz, qxs, qys, qzs, 2)
        nsum = finish4(msrc, offls, px, py, pz, 32)
        msrc, offls = scan4(cx, cy, cz, qxs, qys, qzs, 1)
        esum = finish4(msrc, offls, cx, cy, cz, 4)
        for qq in range(_NQB):
            out = jnp.where(iota == 0, qxs[qq], zf)
            out = jnp.where(iota == 1, qys[qq], out)
            out = jnp.where(iota == 2, qzs[qq], out)
            out = jnp.where(iota == 3, nsum[qq][0], out)
            out = jnp.where(iota == 4, nsum[qq][1], out)
            out = jnp.where(iota == 5, nsum[qq][2], out)
            out = jnp.where(iota == 6, esum[qq][0], out)
            out = jnp.where(iota == 7, esum[qq][1], out)
            out = jnp.where(iota == 8, esum[qq][2], out)
            out_v[pl.ds((j0 + qq) * _OUTW, _OUTW)] = out
        return carry
    lax.fori_loop(0, _QPW // _NQB, qgroup, 0)
    pltpu.sync_copy(out_v, out_hbm.at[pl.ds(wid * _QPW * _OUTW, _QPW * _OUTW)])


_knn_call = functools.partial(
    pl.kernel,
    out_type=jax.ShapeDtypeStruct((_NW * _QPW * _OUTW,), jnp.float32),
    mesh=plsc.VectorSubcoreMesh(core_axis_name="c", subcore_axis_name="s",
                                num_cores=_NC, num_subcores=_NS),
    compiler_params=pltpu.CompilerParams(needs_layout_passes=False),
    scratch_types=[
        pltpu.VMEM((_N,), jnp.float32),
        pltpu.VMEM((_N,), jnp.float32),
        pltpu.VMEM((_N,), jnp.float32),
        pltpu.VMEM((_N,), jnp.float32),
        pltpu.VMEM((_N,), jnp.float32),
        pltpu.VMEM((_N,), jnp.float32),
        pltpu.VMEM((max(_QPW, 128),), jnp.int32),
        pltpu.VMEM((_NQB * _S * _L,), jnp.float32),
        pltpu.VMEM((_NQB * _S * _L,), jnp.int32),
        pltpu.VMEM((_NQB * _S2PAD * _L,), jnp.float32),
        pltpu.VMEM((_NQB * _S2PAD * _L,), jnp.int32),
        pltpu.VMEM((_QPW * _OUTW,), jnp.float32),
    ],
)(_knn_body)


def _mlp_body(sc_ref, w1_ref, b1_ref, w2_ref, b2_ref, afc_ref, anv_ref,
              wfc_ref, wft_ref, bin_ref, wb_ref, bb_ref, wo_ref, bo_ref,
              out_ref):
    f32 = jnp.float32

    def dot(a, bm):
        return lax.dot_general(a, bm, (((1,), (0,)), ((), ())),
                               preferred_element_type=f32)
    sc = sc_ref[...]
    h = jnp.maximum(dot(sc, w1_ref[...]) + b1_ref[...], 0.0)
    feat = dot(h, w2_ref[...]) + b2_ref[...]
    fc = dot(sc, afc_ref[...])
    nv = dot(sc, anv_ref[...])
    s = jnp.maximum(dot(fc, wfc_ref[...]) + dot(feat, wft_ref[...])
                    + bin_ref[...], 0.0)
    for i in range(4):
        s = jnp.maximum(dot(s, wb_ref[i]) + bb_ref[i][None, :], 0.0) + s
    g = dot(s, wo_ref[...]) + bo_ref[...]
    diff = nv + g
    out_ref[...] = (jnp.sum(diff * diff) * f32(0.5 * 100.0 / (_B * _Q))
                    ).reshape(1, 1)


def kernel(pcl_noisy, pcl_clean, params, pnt_idx):
    f32 = jnp.float32
    noisy_flat = jnp.transpose(pcl_noisy, (0, 2, 1)).reshape(-1)
    clean_flat = jnp.transpose(pcl_clean, (0, 2, 1)).reshape(-1)
    sc = _knn_call(noisy_flat, clean_flat,
                   pnt_idx.astype(jnp.int32)).reshape(_B * _Q, _OUTW)

    p = params
    w1 = jnp.zeros((_OUTW, 64), f32).at[0:3].set(p['fW1'])
    b1 = p['fb1'].reshape(1, 64)
    w2 = p['fW2']
    b2 = p['fb2'].reshape(1, 128)
    afc = (jnp.zeros((_OUTW, _OUTW), f32)
           .at[0, 0].set(-1.0).at[1, 1].set(-1.0).at[2, 2].set(-1.0)
           .at[3, 0].set(1.0 / 32).at[4, 1].set(1.0 / 32)
           .at[5, 2].set(1.0 / 32))
    anv = (jnp.zeros((_OUTW, _OUTW), f32)
           .at[0, 0].set(1.0).at[1, 1].set(1.0).at[2, 2].set(1.0)
           .at[6, 0].set(-0.25).at[7, 1].set(-0.25).at[8, 2].set(-0.25))
    wfc = jnp.zeros((_OUTW, 128), f32).at[0:3].set(p['sWin'][0:3])
    wft = p['sWin'][3:]
    bin_ = p['sbin'].reshape(1, 128)
    wb = jnp.stack(p['sWb'])
    bb = jnp.stack(p['sbb'])
    wo = jnp.zeros((128, _OUTW), f32).at[:, 0:3].set(p['sWout'])
    bo = jnp.zeros((1, _OUTW), f32).at[0, 0:3].set(p['sbout'])

    loss = pl.pallas_call(
        _mlp_body,
        out_shape=jax.ShapeDtypeStruct((1, 1), f32),
    )(sc, w1, b1, w2, b2, afc, anv, wfc, wft, bin_, wb, bb, wo, bo)
    return loss[0, 0]
